# Initial kernel scaffold; baseline (speedup 1.0000x reference)
import functools

import jax
import jax.numpy as jnp
from jax import lax
from jax.experimental import pallas as pl
from jax.experimental.pallas import tpu as pltpu
from jax.experimental.pallas import tpu_sc as plsc

B = 8
N = 8192
GROUPS_ = 512
K_ = 32
D_ = 256
SUB = 8
LANE = N // SUB


def _fps_body(f0_ref, xc_ref, cent_ref, nxt_ref, dist_ref):
    b = pl.program_id(0)
    x = xc_ref[0, 0]
    y = xc_ref[0, 1]
    z = xc_ref[0, 2]
    dist_ref[...] = jnp.full((SUB, LANE), 1e10, jnp.float32)
    flat_iota = (lax.broadcasted_iota(jnp.int32, (SUB, LANE), 0) * LANE
                 + lax.broadcasted_iota(jnp.int32, (SUB, LANE), 1))
    iota_g = lax.broadcasted_iota(jnp.int32, (1, GROUPS_), 1)

    def body(i, carry):
        f, acc, nxx, nxy, nxz = carry
        r = f // LANE
        c = f % LANE
        slab = xc_ref[0, :, pl.ds(r, 1), :]
        cpt = lax.dynamic_slice(slab, (0, 0, c), (3, 1, 1))
        cx = cpt[0]
        cy = cpt[1]
        cz = cpt[2]
        dx = x - cx
        dy = y - cy
        dz = z - cz
        d = dx * dx + dy * dy + dz * dz
        dm = jnp.minimum(dist_ref[...], d)
        dist_ref[...] = dm
        m = jnp.max(dm)
        fnew = jnp.min(jnp.where(dm == m, flat_iota, jnp.int32(2**30)))
        sel = iota_g == i
        acc = jnp.where(sel, f, acc)
        nxx = jnp.where(sel, cx, nxx)
        nxy = jnp.where(sel, cy, nxy)
        nxz = jnp.where(sel, cz, nxz)
        return fnew, acc, nxx, nxy, nxz

    f0 = f0_ref[b]
    zi = jnp.zeros((1, GROUPS_), jnp.int32)
    zf = jnp.zeros((1, GROUPS_), jnp.float32)
    _, acc, nxx, nxy, nxz = lax.fori_loop(0, GROUPS_, body,
                                          (f0, zi, zf, zf, zf))
    cent_ref[0] = acc + b * N
    nxt_ref[0, 0:1, :] = nxx
    nxt_ref[0, 1:2, :] = nxy
    nxt_ref[0, 2:3, :] = nxz


def _fps_call(xyz_c, f0, interpret=False):
    return pl.pallas_call(
        _fps_body,
        grid=(B,),
        in_specs=[
            pl.BlockSpec(memory_space=pltpu.SMEM),
            pl.BlockSpec((1, 3, SUB, LANE), lambda b: (b, 0, 0, 0)),
        ],
        out_specs=[
            pl.BlockSpec((1, 1, GROUPS_), lambda b: (b, 0, 0)),
            pl.BlockSpec((1, 3, GROUPS_), lambda b: (b, 0, 0)),
        ],
        out_shape=[
            jax.ShapeDtypeStruct((B, 1, GROUPS_), jnp.int32),
            jax.ShapeDtypeStruct((B, 3, GROUPS_), jnp.float32),
        ],
        scratch_shapes=[pltpu.VMEM((SUB, LANE), jnp.float32)],
        interpret=interpret,
    )(f0, xyz_c)


QB = 128


def _knn_body(q_ref, xt_ref, out_ref, dist_ref):
    b = pl.program_id(0)
    q = q_ref[0]
    x_t = xt_ref[0]
    qx = q[:, 0:1]
    qy = q[:, 1:2]
    qz = q[:, 2:3]
    mm = lax.dot_general(q, x_t, (((1,), (0,)), ((), ())),
                         preferred_element_type=jnp.float32)
    s2 = qx * qx + qy * qy + qz * qz
    x0 = x_t[0:1, :]
    x1 = x_t[1:2, :]
    x2r = x_t[2:3, :]
    x2 = x0 * x0 + x1 * x1 + x2r * x2r
    d = -2.0 * mm
    d = d + s2
    d = d + x2
    dist_ref[...] = d
    ci = lax.broadcasted_iota(jnp.int32, (1, N), 1)
    ki = lax.broadcasted_iota(jnp.int32, (1, K_), 1)

    def body(k, idxmat):
        dd = dist_ref[...]
        m = jnp.min(dd, axis=1, keepdims=True)
        idxk = jnp.min(jnp.where(dd == m, ci, jnp.int32(2**30)),
                       axis=1, keepdims=True)
        dist_ref[...] = jnp.where(ci == idxk, jnp.float32(jnp.inf), dd)
        return jnp.where(ki == k, idxk, idxmat)

    idxmat = lax.fori_loop(0, K_, body, jnp.zeros((QB, K_), jnp.int32))
    out_ref[0] = idxmat + b * N


def _knn_call(new_xyz, xyz_t, interpret=False):
    return pl.pallas_call(
        _knn_body,
        grid=(B, GROUPS_ // QB),
        in_specs=[
            pl.BlockSpec((1, QB, 3), lambda b, q: (b, q, 0)),
            pl.BlockSpec((1, 3, N), lambda b, q: (b, 0, 0)),
        ],
        out_specs=pl.BlockSpec((1, QB, K_), lambda b, q: (b, q, 0)),
        out_shape=jax.ShapeDtypeStruct((B, GROUPS_, K_), jnp.int32),
        scratch_shapes=[pltpu.VMEM((QB, N), jnp.float32)],
        interpret=interpret,
    )(new_xyz, xyz_t)


def _sc_gather(table, idx, chunk):
    rows, d = table.shape
    nidx = idx.shape[0]
    info = plsc.get_sparse_core_info()
    nw = info.num_cores * info.num_subcores
    per_w = nidx // nw
    nchunks = per_w // chunk
    mesh = plsc.VectorSubcoreMesh(core_axis_name="c", subcore_axis_name="s")

    @functools.partial(
        pl.kernel,
        out_type=jax.ShapeDtypeStruct((nidx, d), jnp.float32),
        mesh=mesh,
        scratch_types=[
            pltpu.VMEM((per_w,), jnp.int32),
            pltpu.VMEM((chunk, d), jnp.float32),
            pltpu.SemaphoreType.DMA,
        ],
    )
    def k(table_hbm, idx_hbm, out_hbm, idx_v, rows_v, sem):
        wid = lax.axis_index("s") * info.num_cores + lax.axis_index("c")
        base = wid * per_w
        pltpu.sync_copy(idx_hbm.at[pl.ds(base, per_w)], idx_v)

        def body(c, _):
            off = c * chunk
            pltpu.async_copy(
                table_hbm.at[idx_v.at[pl.ds(off, chunk)]], rows_v, sem
            ).wait()
            pltpu.sync_copy(rows_v, out_hbm.at[pl.ds(base + off, chunk)])
            return 0

        lax.fori_loop(0, nchunks, body, 0)

    return k(table, idx)


def kernel(xyz, points):
    xyz_t = jnp.transpose(xyz, (0, 2, 1))
    xyz_c = xyz_t.reshape(B, 3, SUB, LANE)
    f0 = jax.random.randint(jax.random.key(42), (B,), 0, N).astype(jnp.int32)

    cent, nxt = _fps_call(xyz_c, f0)
    fps_flat = cent.reshape(B * GROUPS_)
    new_xyz = jnp.transpose(nxt, (0, 2, 1))

    idx = _knn_call(new_xyz, xyz_t)
    idx_flat = idx.reshape(B * GROUPS_ * K_)

    points_flat = points.reshape(B * N, D_)
    xyz16 = jnp.pad(xyz.reshape(B * N, 3), ((0, 0), (0, 13)))

    new_points = _sc_gather(points_flat, fps_flat, 128)
    grouped_points = _sc_gather(points_flat, idx_flat, 128)
    grouped_xyz16 = _sc_gather(xyz16, idx_flat, 512)

    return (
        new_xyz,
        new_points.reshape(B, GROUPS_, D_),
        grouped_xyz16[:, :3].reshape(B, GROUPS_, K_, 3),
        grouped_points.reshape(B, GROUPS_, K_, D_),
    )


# speedup vs baseline: 6.2014x; 6.2014x over previous
import functools

import jax
import jax.numpy as jnp
from jax import lax
from jax.experimental import pallas as pl
from jax.experimental.pallas import tpu as pltpu
from jax.experimental.pallas import tpu_sc as plsc

B = 8
N = 8192
GROUPS_ = 512
K_ = 32
D_ = 256
SUB = 8
LANE = N // SUB


def _fps_body(f0_ref, xc_ref, cent_ref, nxt_ref, dist_ref):
    b = pl.program_id(0)
    x = xc_ref[0, 0]
    y = xc_ref[0, 1]
    z = xc_ref[0, 2]
    dist_ref[...] = jnp.full((SUB, LANE), 1e10, jnp.float32)
    flat_iota = (lax.broadcasted_iota(jnp.int32, (SUB, LANE), 0) * LANE
                 + lax.broadcasted_iota(jnp.int32, (SUB, LANE), 1))
    iota_g = lax.broadcasted_iota(jnp.int32, (1, GROUPS_), 1)
    iota_l = lax.broadcasted_iota(jnp.int32, (1, LANE), 1)

    def body(i, carry):
        f, acc, nxx, nxy, nxz = carry
        r = f // LANE
        c = f % LANE
        lsel = iota_l == c
        row_x = xc_ref[0, 0, pl.ds(r, 1), :]
        row_y = xc_ref[0, 1, pl.ds(r, 1), :]
        row_z = xc_ref[0, 2, pl.ds(r, 1), :]
        cx = jnp.sum(jnp.where(lsel, row_x, 0.0), axis=1, keepdims=True)
        cy = jnp.sum(jnp.where(lsel, row_y, 0.0), axis=1, keepdims=True)
        cz = jnp.sum(jnp.where(lsel, row_z, 0.0), axis=1, keepdims=True)
        dx = x - cx
        dy = y - cy
        dz = z - cz
        d = dx * dx + dy * dy + dz * dz
        dm = jnp.minimum(dist_ref[...], d)
        dist_ref[...] = dm
        m = jnp.max(dm)
        fnew = jnp.min(jnp.where(dm == m, flat_iota, jnp.int32(2**30)))
        sel = iota_g == i
        acc = jnp.where(sel, f, acc)
        nxx = jnp.where(sel, cx, nxx)
        nxy = jnp.where(sel, cy, nxy)
        nxz = jnp.where(sel, cz, nxz)
        return fnew, acc, nxx, nxy, nxz

    f0 = f0_ref[b]
    zi = jnp.zeros((1, GROUPS_), jnp.int32)
    zf = jnp.zeros((1, GROUPS_), jnp.float32)
    _, acc, nxx, nxy, nxz = lax.fori_loop(0, GROUPS_, body,
                                          (f0, zi, zf, zf, zf))
    cent_ref[0] = acc + b * N
    nxt_ref[0, 0:1, :] = nxx
    nxt_ref[0, 1:2, :] = nxy
    nxt_ref[0, 2:3, :] = nxz


def _fps_call(xyz_c, f0, interpret=False):
    return pl.pallas_call(
        _fps_body,
        grid=(B,),
        in_specs=[
            pl.BlockSpec(memory_space=pltpu.SMEM),
            pl.BlockSpec((1, 3, SUB, LANE), lambda b: (b, 0, 0, 0)),
        ],
        out_specs=[
            pl.BlockSpec((1, 1, GROUPS_), lambda b: (b, 0, 0)),
            pl.BlockSpec((1, 3, GROUPS_), lambda b: (b, 0, 0)),
        ],
        out_shape=[
            jax.ShapeDtypeStruct((B, 1, GROUPS_), jnp.int32),
            jax.ShapeDtypeStruct((B, 3, GROUPS_), jnp.float32),
        ],
        scratch_shapes=[pltpu.VMEM((SUB, LANE), jnp.float32)],
        interpret=interpret,
    )(f0, xyz_c)


QB = 128


def _knn_body(q_ref, xt_ref, out_ref, dist_ref):
    b = pl.program_id(0)
    q = q_ref[0]
    x_t = xt_ref[0]
    qx = q[:, 0:1]
    qy = q[:, 1:2]
    qz = q[:, 2:3]
    mm = lax.dot_general(q, x_t, (((1,), (0,)), ((), ())),
                         preferred_element_type=jnp.float32)
    s2 = qx * qx + qy * qy + qz * qz
    x0 = x_t[0:1, :]
    x1 = x_t[1:2, :]
    x2r = x_t[2:3, :]
    x2 = x0 * x0 + x1 * x1 + x2r * x2r
    d = -2.0 * mm
    d = d + s2
    d = d + x2
    dist_ref[...] = d
    ci = lax.broadcasted_iota(jnp.int32, (1, N), 1)
    ki = lax.broadcasted_iota(jnp.int32, (1, K_), 1)

    def body(k, idxmat):
        dd = dist_ref[...]
        m = jnp.min(dd, axis=1, keepdims=True)
        idxk = jnp.min(jnp.where(dd == m, ci, jnp.int32(2**30)),
                       axis=1, keepdims=True)
        dist_ref[...] = jnp.where(ci == idxk, jnp.float32(jnp.inf), dd)
        return jnp.where(ki == k, idxk, idxmat)

    idxmat = lax.fori_loop(0, K_, body, jnp.zeros((QB, K_), jnp.int32))
    out_ref[0] = idxmat + b * N


def _knn_call(new_xyz, xyz_t, interpret=False):
    return pl.pallas_call(
        _knn_body,
        grid=(B, GROUPS_ // QB),
        in_specs=[
            pl.BlockSpec((1, QB, 3), lambda b, q: (b, q, 0)),
            pl.BlockSpec((1, 3, N), lambda b, q: (b, 0, 0)),
        ],
        out_specs=pl.BlockSpec((1, QB, K_), lambda b, q: (b, q, 0)),
        out_shape=jax.ShapeDtypeStruct((B, GROUPS_, K_), jnp.int32),
        scratch_shapes=[pltpu.VMEM((QB, N), jnp.float32)],
        interpret=interpret,
    )(new_xyz, xyz_t)


def _sc_gather(table, idx, chunk, tc_tiling=True):
    rows, d = table.shape
    nidx = idx.shape[0]
    info = plsc.get_sparse_core_info()
    nw = info.num_cores * info.num_subcores
    per_w = nidx // nw
    nchunks = per_w // chunk
    mesh = plsc.VectorSubcoreMesh(core_axis_name="c", subcore_axis_name="s")

    @functools.partial(
        pl.kernel,
        out_type=jax.ShapeDtypeStruct((nidx, d), jnp.float32),
        mesh=mesh,
        compiler_params=pltpu.CompilerParams(use_tc_tiling_on_sc=tc_tiling),
        scratch_types=[
            pltpu.VMEM((per_w,), jnp.int32),
            pltpu.VMEM((chunk, d), jnp.float32),
            pltpu.SemaphoreType.DMA,
        ],
    )
    def k(table_hbm, idx_hbm, out_hbm, idx_v, rows_v, sem):
        wid = lax.axis_index("s") * info.num_cores + lax.axis_index("c")
        base = wid * per_w
        pltpu.sync_copy(idx_hbm.at[pl.ds(base, per_w)], idx_v)

        def body(c, _):
            off = c * chunk
            pltpu.async_copy(
                table_hbm.at[idx_v.at[pl.ds(off, chunk)]], rows_v, sem
            ).wait()
            pltpu.sync_copy(rows_v, out_hbm.at[pl.ds(base + off, chunk)])
            return 0

        lax.fori_loop(0, nchunks, body, 0)

    return k(table, idx)


def kernel(xyz, points):
    xyz_t = jnp.transpose(xyz, (0, 2, 1))
    xyz_c = xyz_t.reshape(B, 3, SUB, LANE)
    f0 = jax.random.randint(jax.random.key(42), (B,), 0, N).astype(jnp.int32)

    cent, nxt = _fps_call(xyz_c, f0)
    fps_flat = cent.reshape(B * GROUPS_)
    new_xyz = jnp.transpose(nxt, (0, 2, 1))

    idx = _knn_call(new_xyz, xyz_t)
    idx_flat = idx.reshape(B * GROUPS_ * K_)

    points_flat = points.reshape(B * N, D_)
    xyz16 = jnp.pad(xyz.reshape(B * N, 3), ((0, 0), (0, 13)))

    new_points = _sc_gather(points_flat, fps_flat, 128)
    grouped_points = _sc_gather(points_flat, idx_flat, 128)
    grouped_xyz16 = _sc_gather(xyz16, idx_flat, 512, tc_tiling=False)

    return (
        new_xyz,
        new_points.reshape(B, GROUPS_, D_),
        grouped_xyz16[:, :3].reshape(B, GROUPS_, K_, 3),
        grouped_points.reshape(B, GROUPS_, K_, D_),
    )


# speedup vs baseline: 8.1586x; 1.3156x over previous
import functools

import jax
import jax.numpy as jnp
from jax import lax
from jax.experimental import pallas as pl
from jax.experimental.pallas import tpu as pltpu
from jax.experimental.pallas import tpu_sc as plsc

B = 8
N = 8192
GROUPS_ = 512
K_ = 32
D_ = 256
SUB = 8
LANE = N // SUB


def _fps_body(f0_ref, xc_ref, cent_ref, dist_ref):
    for b in range(B):
        dist_ref[b] = jnp.full((SUB, LANE), 1e10, jnp.float32)
    flat_iota = (lax.broadcasted_iota(jnp.int32, (SUB, LANE), 0) * LANE
                 + lax.broadcasted_iota(jnp.int32, (SUB, LANE), 1))
    iota_g = lax.broadcasted_iota(jnp.int32, (1, GROUPS_), 1)
    iota_l = lax.broadcasted_iota(jnp.int32, (1, LANE), 1)

    def body(i, carry):
        fs, accs = carry
        sel = iota_g == i
        nfs, naccs = [], []
        for b in range(B):
            f = fs[b]
            r = f // LANE
            c = f % LANE
            lsel = iota_l == c
            row_x = xc_ref[b, 0, pl.ds(r, 1), :]
            row_y = xc_ref[b, 1, pl.ds(r, 1), :]
            row_z = xc_ref[b, 2, pl.ds(r, 1), :]
            cx = jnp.sum(jnp.where(lsel, row_x, 0.0), axis=1, keepdims=True)
            cy = jnp.sum(jnp.where(lsel, row_y, 0.0), axis=1, keepdims=True)
            cz = jnp.sum(jnp.where(lsel, row_z, 0.0), axis=1, keepdims=True)
            dx = xc_ref[b, 0] - cx
            dy = xc_ref[b, 1] - cy
            dz = xc_ref[b, 2] - cz
            d = dx * dx + dy * dy + dz * dz
            dm = jnp.minimum(dist_ref[b], d)
            dist_ref[b] = dm
            m = jnp.max(dm)
            nfs.append(jnp.min(jnp.where(dm == m, flat_iota,
                                         jnp.int32(2**30))))
            naccs.append(jnp.where(sel, f, accs[b]))
        return tuple(nfs), tuple(naccs)

    f0s = tuple(f0_ref[b] for b in range(B))
    zi = jnp.zeros((1, GROUPS_), jnp.int32)
    _, accs = lax.fori_loop(0, GROUPS_, body, (f0s, (zi,) * B))
    for b in range(B):
        cent_ref[b] = accs[b] + b * N


def _fps_call(xyz_c, f0, interpret=False):
    return pl.pallas_call(
        _fps_body,
        in_specs=[
            pl.BlockSpec(memory_space=pltpu.SMEM),
            pl.BlockSpec((B, 3, SUB, LANE), lambda: (0, 0, 0, 0)),
        ],
        out_specs=pl.BlockSpec((B, 1, GROUPS_), lambda: (0, 0, 0)),
        out_shape=jax.ShapeDtypeStruct((B, 1, GROUPS_), jnp.int32),
        scratch_shapes=[pltpu.VMEM((B, SUB, LANE), jnp.float32)],
        interpret=interpret,
    )(f0, xyz_c)


QB = 128


def _knn_body(q_ref, xt_ref, out_ref, dist_ref):
    b = pl.program_id(0)
    q = q_ref[0]
    x_t = xt_ref[0]
    qx = q[:, 0:1]
    qy = q[:, 1:2]
    qz = q[:, 2:3]
    mm = lax.dot_general(q, x_t, (((1,), (0,)), ((), ())),
                         preferred_element_type=jnp.float32)
    s2 = qx * qx + qy * qy + qz * qz
    x0 = x_t[0:1, :]
    x1 = x_t[1:2, :]
    x2r = x_t[2:3, :]
    x2 = x0 * x0 + x1 * x1 + x2r * x2r
    d = -2.0 * mm
    d = d + s2
    d = d + x2
    dist_ref[...] = d
    ci = lax.broadcasted_iota(jnp.int32, (1, N), 1)
    ki = lax.broadcasted_iota(jnp.int32, (1, K_), 1)

    def body(k, idxmat):
        dd = dist_ref[...]
        m = jnp.min(dd, axis=1, keepdims=True)
        idxk = jnp.min(jnp.where(dd == m, ci, jnp.int32(2**30)),
                       axis=1, keepdims=True)
        dist_ref[...] = jnp.where(ci == idxk, jnp.float32(jnp.inf), dd)
        return jnp.where(ki == k, idxk, idxmat)

    idxmat = lax.fori_loop(0, K_, body, jnp.zeros((QB, K_), jnp.int32))
    out_ref[0] = idxmat + b * N


def _knn_call(new_xyz, xyz_t, interpret=False):
    return pl.pallas_call(
        _knn_body,
        grid=(B, GROUPS_ // QB),
        in_specs=[
            pl.BlockSpec((1, QB, 3), lambda b, q: (b, q, 0)),
            pl.BlockSpec((1, 3, N), lambda b, q: (b, 0, 0)),
        ],
        out_specs=pl.BlockSpec((1, QB, K_), lambda b, q: (b, q, 0)),
        out_shape=jax.ShapeDtypeStruct((B, GROUPS_, K_), jnp.int32),
        scratch_shapes=[pltpu.VMEM((QB, N), jnp.float32)],
        interpret=interpret,
    )(new_xyz, xyz_t)


def _sc_gather(table, idx, chunk, tc_tiling=True):
    rows, d = table.shape
    nidx = idx.shape[0]
    info = plsc.get_sparse_core_info()
    nw = info.num_cores * info.num_subcores
    per_w = nidx // nw
    nchunks = per_w // chunk
    mesh = plsc.VectorSubcoreMesh(core_axis_name="c", subcore_axis_name="s")

    @functools.partial(
        pl.kernel,
        out_type=jax.ShapeDtypeStruct((nidx, d), jnp.float32),
        mesh=mesh,
        compiler_params=pltpu.CompilerParams(use_tc_tiling_on_sc=tc_tiling),
        scratch_types=[
            pltpu.VMEM((per_w,), jnp.int32),
            pltpu.VMEM((chunk, d), jnp.float32),
            pltpu.SemaphoreType.DMA,
        ],
    )
    def k(table_hbm, idx_hbm, out_hbm, idx_v, rows_v, sem):
        wid = lax.axis_index("s") * info.num_cores + lax.axis_index("c")
        base = wid * per_w
        pltpu.sync_copy(idx_hbm.at[pl.ds(base, per_w)], idx_v)

        def body(c, _):
            off = c * chunk
            pltpu.async_copy(
                table_hbm.at[idx_v.at[pl.ds(off, chunk)]], rows_v, sem
            ).wait()
            pltpu.sync_copy(rows_v, out_hbm.at[pl.ds(base + off, chunk)])
            return 0

        lax.fori_loop(0, nchunks, body, 0)

    return k(table, idx)


def kernel(xyz, points):
    xyz_t = jnp.transpose(xyz, (0, 2, 1))
    xyz_c = xyz_t.reshape(B, 3, SUB, LANE)
    f0 = jax.random.randint(jax.random.key(42), (B,), 0, N).astype(jnp.int32)

    cent = _fps_call(xyz_c, f0)
    fps_flat = cent.reshape(B * GROUPS_)

    points_flat = points.reshape(B * N, D_)
    xyz16 = jnp.pad(xyz.reshape(B * N, 3), ((0, 0), (0, 13)))

    new_xyz16 = _sc_gather(xyz16, fps_flat, 128, tc_tiling=False)
    new_xyz = new_xyz16[:, :3].reshape(B, GROUPS_, 3)

    idx = _knn_call(new_xyz, xyz_t)
    idx_flat = idx.reshape(B * GROUPS_ * K_)

    new_points = _sc_gather(points_flat, fps_flat, 128)
    grouped_points = _sc_gather(points_flat, idx_flat, 128)
    grouped_xyz16 = _sc_gather(xyz16, idx_flat, 512, tc_tiling=False)

    return (
        new_xyz,
        new_points.reshape(B, GROUPS_, D_),
        grouped_xyz16[:, :3].reshape(B, GROUPS_, K_, 3),
        grouped_points.reshape(B, GROUPS_, K_, D_),
    )
